# baseline (speedup 1.0000x reference)
import jax
import jax.numpy as jnp
from jax import lax
from jax.experimental import pallas as pl
from jax.experimental.pallas import tpu as pltpu
from jax.experimental.pallas import tpu_sc as plsc

N_NODES = 10000
N_EDGES = 320000
D = 128

NC = 2
NS = 16
NW = NC * NS
BATCH = 125
NB = 80
CH = 40
NCH = NB // CH
ACC_ROWS = 10112
ZPT = ACC_ROWS // NS
NBUF = 2


def _sc_agg_body(y, srcm, dstm, zeros, out, srcc, dstv,
                 r0, r1, acc, g0, g1, s0, s1):
    rows = (r0, r1)
    gsem = (g0, g1)
    ssem = (s0, s1)
    c = lax.axis_index("c")
    s = lax.axis_index("s")
    wid = c * NS + s
    base = wid * NB
    cp_d = pltpu.async_copy(dstm.at[pl.ds(base, NB)], dstv, g0)
    cp_z = pltpu.async_copy(zeros, acc.at[pl.ds(s * ZPT, ZPT)], g1)
    cp_d.wait()
    cp_z.wait()
    plsc.subcore_barrier()

    def chunk(k, carry):
        kb = pl.multiple_of(k * CH, CH)
        pltpu.sync_copy(srcm.at[pl.ds(base + kb, CH)], srcc)
        pltpu.async_copy(y.at[srcc.at[0]], rows[0], gsem[0])

        def body(i, carry2):
            b = i * NBUF
            for j in range(NBUF):
                jl = b + j

                @pl.when(jl + 1 < CH)
                def _ahead(jl=jl, j=j):
                    pltpu.async_copy(y.at[srcc.at[jl + 1]],
                                     rows[1 - j], gsem[1 - j])

                pltpu.make_async_copy(y.at[srcc.at[jl]], rows[j],
                                      gsem[j]).wait()
                pltpu.sync_copy(rows[j], acc.at[dstv.at[kb + jl]], add=True)
            return carry2

        lax.fori_loop(0, CH // NBUF, body, 0)
        return carry

    lax.fori_loop(0, NCH, chunk, 0)
    plsc.subcore_barrier()
    pltpu.sync_copy(acc.at[pl.ds(s * ZPT, ZPT)],
                    out.at[c, pl.ds(s * ZPT, ZPT)])


_sc_agg = pl.kernel(
    _sc_agg_body,
    out_type=jax.ShapeDtypeStruct((NC, ACC_ROWS, D), jnp.float32),
    mesh=plsc.VectorSubcoreMesh(core_axis_name="c", subcore_axis_name="s"),
    scratch_types=(
        [pltpu.VMEM((CH, BATCH), jnp.int32),
         pltpu.VMEM((NB, BATCH), jnp.int32)]
        + [pltpu.VMEM((BATCH, D), jnp.float32) for _ in range(NBUF)]
        + [pltpu.VMEM_SHARED((ACC_ROWS, D), jnp.float32)]
        + [pltpu.SemaphoreType.DMA for _ in range(2 * NBUF)]
    ),
)


def _mm_body(x_ref, w_ref, o_ref):
    o_ref[...] = jnp.dot(x_ref[...], w_ref[...],
                         preferred_element_type=jnp.float32)


_mm1 = pl.pallas_call(
    _mm_body,
    grid=(2,),
    in_specs=[pl.BlockSpec((5000, D), lambda i: (i, 0)),
              pl.BlockSpec((D, D), lambda i: (0, 0))],
    out_specs=pl.BlockSpec((5000, D), lambda i: (i, 0)),
    out_shape=jax.ShapeDtypeStruct((N_NODES, D), jnp.float32),
)


def _fuse_body(p_ref, b_ref, w_ref, o_ref):
    h = jnp.maximum(p_ref[0] + p_ref[1] + b_ref[...], 0.0)
    o_ref[...] = jnp.dot(h, w_ref[...], preferred_element_type=jnp.float32)


_fuse2 = pl.pallas_call(
    _fuse_body,
    grid=(2,),
    in_specs=[pl.BlockSpec((NC, 5000, D), lambda i: (0, i, 0)),
              pl.BlockSpec((1, D), lambda i: (0, 0)),
              pl.BlockSpec((D, D), lambda i: (0, 0))],
    out_specs=pl.BlockSpec((5000, D), lambda i: (i, 0)),
    out_shape=jax.ShapeDtypeStruct((N_NODES, D), jnp.float32),
)


def _head_body(p_ref, b2_ref, desc_ref, l1wa_ref, l1wb_ref, l1b_ref,
               l2w_ref, l2b_ref, cw_ref, cb_ref, o_ref, acc_ref):
    g = pl.program_id(0)

    @pl.when(g == 0)
    def _init():
        acc_ref[...] = jnp.zeros_like(acc_ref)

    h = jnp.maximum(p_ref[0] + p_ref[1] + b2_ref[...], 0.0)
    acc_ref[...] += jnp.sum(h, axis=0, keepdims=True)

    @pl.when(g == pl.num_programs(0) - 1)
    def _finish():
        hg = acc_ref[...] * (1.0 / N_NODES)
        t = (jnp.dot(hg, l1wa_ref[...], preferred_element_type=jnp.float32)
             + jnp.dot(desc_ref[...], l1wb_ref[...],
                       preferred_element_type=jnp.float32)
             + l1b_ref[...])
        t = jnp.maximum(t, 0.0)
        t = jnp.maximum(
            jnp.dot(t, l2w_ref[...], preferred_element_type=jnp.float32)
            + l2b_ref[...], 0.0)
        o_ref[...] = (jnp.dot(t, cw_ref[...],
                              preferred_element_type=jnp.float32)
                      + cb_ref[...])


_head = pl.pallas_call(
    _head_body,
    grid=(5,),
    in_specs=[pl.BlockSpec((NC, 2000, D), lambda i: (0, i, 0)),
              pl.BlockSpec((1, D), lambda i: (0, 0)),
              pl.BlockSpec((1, 16), lambda i: (0, 0)),
              pl.BlockSpec((D, 500), lambda i: (0, 0)),
              pl.BlockSpec((16, 500), lambda i: (0, 0)),
              pl.BlockSpec((1, 500), lambda i: (0, 0)),
              pl.BlockSpec((500, 100), lambda i: (0, 0)),
              pl.BlockSpec((1, 100), lambda i: (0, 0)),
              pl.BlockSpec((100, 10), lambda i: (0, 0)),
              pl.BlockSpec((1, 10), lambda i: (0, 0))],
    out_specs=pl.BlockSpec((1, 10), lambda i: (0, 0)),
    out_shape=jax.ShapeDtypeStruct((1, 10), jnp.float32),
    scratch_shapes=[pltpu.VMEM((1, D), jnp.float32)],
)


def kernel(features, edge_index, descriptors,
           W1, b1, W2, b2, L1w, L1b, L2w, L2b, Cw, Cb):
    ei = edge_index.astype(jnp.int32)
    srcm = ei[0].reshape(NW * NB, BATCH)
    dstm = ei[1].reshape(NW * NB, BATCH)
    zeros = jnp.zeros((ZPT, D), jnp.float32)

    y1 = _mm1(features, W1)
    p1 = _sc_agg(y1, srcm, dstm, zeros)
    y2 = _fuse2(p1, b1.reshape(1, D), W2)
    p2 = _sc_agg(y2, srcm, dstm, zeros)
    return _head(p2, b2.reshape(1, D), descriptors,
                 L1w[:D], L1w[D:], L1b.reshape(1, 500),
                 L2w, L2b.reshape(1, 100), Cw, Cb.reshape(1, 10))


# speedup vs baseline: 1.0033x; 1.0033x over previous
import jax
import jax.numpy as jnp
from jax import lax
from jax.experimental import pallas as pl
from jax.experimental.pallas import tpu as pltpu
from jax.experimental.pallas import tpu_sc as plsc

N_NODES = 10000
N_EDGES = 320000
D = 128

NC = 2
NS = 16
NW = NC * NS
BATCH = 125
NB = 80
CH = 40
NCH = NB // CH
ACC_ROWS = 10112
ZPT = ACC_ROWS // NS
NBUF = 2


def _sc_agg_body(y, srcm, dstm, zeros, out, srcc, dstv,
                 r0, r1, acc, g0, g1):
    rows = (r0, r1)
    gsem = (g0, g1)
    c = lax.axis_index("c")
    s = lax.axis_index("s")
    wid = c * NS + s
    base = wid * NB
    cp_d = pltpu.async_copy(dstm.at[pl.ds(base, NB)], dstv, g0)
    cp_z = pltpu.async_copy(zeros, acc.at[pl.ds(s * ZPT, ZPT)], g1)
    cp_d.wait()
    cp_z.wait()
    plsc.subcore_barrier()

    def chunk(k, carry):
        kb = pl.multiple_of(k * CH, CH)
        pltpu.sync_copy(srcm.at[pl.ds(base + kb, CH)], srcc)
        pltpu.async_copy(y.at[srcc.at[0]], rows[0], gsem[0])

        def body(i, carry2):
            b = i * NBUF
            for j in range(NBUF):
                jl = b + j

                @pl.when(jl + 1 < CH)
                def _ahead(jl=jl, j=j):
                    pltpu.async_copy(y.at[srcc.at[jl + 1]],
                                     rows[1 - j], gsem[1 - j])

                pltpu.make_async_copy(y.at[srcc.at[jl]], rows[j],
                                      gsem[j]).wait()
                pltpu.sync_copy(rows[j], acc.at[dstv.at[kb + jl]], add=True)
            return carry2

        lax.fori_loop(0, CH // NBUF, body, 0)
        return carry

    lax.fori_loop(0, NCH, chunk, 0)
    plsc.subcore_barrier()
    pltpu.sync_copy(acc.at[pl.ds(s * ZPT, ZPT)],
                    out.at[c, pl.ds(s * ZPT, ZPT)])


_sc_agg = pl.kernel(
    _sc_agg_body,
    out_type=jax.ShapeDtypeStruct((NC, ACC_ROWS, D), jnp.float32),
    mesh=plsc.VectorSubcoreMesh(core_axis_name="c", subcore_axis_name="s"),
    scratch_types=(
        [pltpu.VMEM((CH, BATCH), jnp.int32),
         pltpu.VMEM((NB, BATCH), jnp.int32)]
        + [pltpu.VMEM((BATCH, D), jnp.float32) for _ in range(NBUF)]
        + [pltpu.VMEM_SHARED((ACC_ROWS, D), jnp.float32)]
        + [pltpu.SemaphoreType.DMA for _ in range(NBUF)]
    ),
)


def _mm_body(x_ref, w_ref, o_ref):
    o_ref[...] = jnp.dot(x_ref[...], w_ref[...],
                         preferred_element_type=jnp.float32)


_mm1 = pl.pallas_call(
    _mm_body,
    grid=(2,),
    in_specs=[pl.BlockSpec((5000, D), lambda i: (i, 0)),
              pl.BlockSpec((D, D), lambda i: (0, 0))],
    out_specs=pl.BlockSpec((5000, D), lambda i: (i, 0)),
    out_shape=jax.ShapeDtypeStruct((N_NODES, D), jnp.float32),
)


def _fuse_body(p_ref, b_ref, w_ref, o_ref):
    h = jnp.maximum(p_ref[0] + p_ref[1] + b_ref[...], 0.0)
    o_ref[...] = jnp.dot(h, w_ref[...], preferred_element_type=jnp.float32)


_fuse2 = pl.pallas_call(
    _fuse_body,
    grid=(2,),
    in_specs=[pl.BlockSpec((NC, 5000, D), lambda i: (0, i, 0)),
              pl.BlockSpec((1, D), lambda i: (0, 0)),
              pl.BlockSpec((D, D), lambda i: (0, 0))],
    out_specs=pl.BlockSpec((5000, D), lambda i: (i, 0)),
    out_shape=jax.ShapeDtypeStruct((N_NODES, D), jnp.float32),
)


def _head_body(p_ref, b2_ref, desc_ref, l1wa_ref, l1wb_ref, l1b_ref,
               l2w_ref, l2b_ref, cw_ref, cb_ref, o_ref, acc_ref):
    g = pl.program_id(0)

    @pl.when(g == 0)
    def _init():
        acc_ref[...] = jnp.zeros_like(acc_ref)

    h = jnp.maximum(p_ref[0] + p_ref[1] + b2_ref[...], 0.0)
    acc_ref[...] += jnp.sum(h, axis=0, keepdims=True)

    @pl.when(g == pl.num_programs(0) - 1)
    def _finish():
        hg = acc_ref[...] * (1.0 / N_NODES)
        t = (jnp.dot(hg, l1wa_ref[...], preferred_element_type=jnp.float32)
             + jnp.dot(desc_ref[...], l1wb_ref[...],
                       preferred_element_type=jnp.float32)
             + l1b_ref[...])
        t = jnp.maximum(t, 0.0)
        t = jnp.maximum(
            jnp.dot(t, l2w_ref[...], preferred_element_type=jnp.float32)
            + l2b_ref[...], 0.0)
        o_ref[...] = (jnp.dot(t, cw_ref[...],
                              preferred_element_type=jnp.float32)
                      + cb_ref[...])


_head = pl.pallas_call(
    _head_body,
    grid=(5,),
    in_specs=[pl.BlockSpec((NC, 2000, D), lambda i: (0, i, 0)),
              pl.BlockSpec((1, D), lambda i: (0, 0)),
              pl.BlockSpec((1, 16), lambda i: (0, 0)),
              pl.BlockSpec((D, 500), lambda i: (0, 0)),
              pl.BlockSpec((16, 500), lambda i: (0, 0)),
              pl.BlockSpec((1, 500), lambda i: (0, 0)),
              pl.BlockSpec((500, 100), lambda i: (0, 0)),
              pl.BlockSpec((1, 100), lambda i: (0, 0)),
              pl.BlockSpec((100, 10), lambda i: (0, 0)),
              pl.BlockSpec((1, 10), lambda i: (0, 0))],
    out_specs=pl.BlockSpec((1, 10), lambda i: (0, 0)),
    out_shape=jax.ShapeDtypeStruct((1, 10), jnp.float32),
    scratch_shapes=[pltpu.VMEM((1, D), jnp.float32)],
)


def kernel(features, edge_index, descriptors,
           W1, b1, W2, b2, L1w, L1b, L2w, L2b, Cw, Cb):
    ei = edge_index.astype(jnp.int32)
    srcm = ei[0].reshape(NW * NB, BATCH)
    dstm = ei[1].reshape(NW * NB, BATCH)
    zeros = jnp.zeros((ZPT, D), jnp.float32)

    y1 = _mm1(features, W1)
    p1 = _sc_agg(y1, srcm, dstm, zeros)
    y2 = _fuse2(p1, b1.reshape(1, D), W2)
    p2 = _sc_agg(y2, srcm, dstm, zeros)
    return _head(p2, b2.reshape(1, D), descriptors,
                 L1w[:D], L1w[D:], L1b.reshape(1, 500),
                 L2w, L2b.reshape(1, 100), Cw, Cb.reshape(1, 10))
